# baseline (speedup 1.0000x reference)
import functools

import jax
import jax.numpy as jnp
from jax import lax
from jax.experimental import pallas as pl
from jax.experimental.pallas import tpu as pltpu
from jax.experimental.pallas import tpu_sc as plsc

VOCAB = 1000
D_MODEL = 128
BIGRAM_VOCAB = 1000000
BIGRAM_DIM = 64
B, S = 1024, 20
N = B * S
MOD = BIGRAM_VOCAB - 1

CHUNK = 128
LANES = 16

_INFO = plsc.get_sparse_core_info()
_NW = _INFO.num_cores * _INFO.num_subcores
BW = B // _NW
PER_W = BW * S
_MESH = dict(core_axis_name="c", subcore_axis_name="s")


def _stage_tokens(tokt_hbm, tokv, b0):
    for s in range(S):
        pltpu.sync_copy(tokt_hbm.at[s, pl.ds(b0, BW)],
                        tokv.at[pl.ds(s * BW, BW)])


def _sc_big(tokens_t, big_emb):
    @functools.partial(
        pl.kernel,
        mesh=plsc.VectorSubcoreMesh(**_MESH),
        out_type=jax.ShapeDtypeStruct((N, BIGRAM_DIM), jnp.float32),
        scratch_types=[
            pltpu.VMEM((S * BW,), jnp.int32),
            pltpu.VMEM((PER_W,), jnp.int32),
            pltpu.VMEM((PER_W, BIGRAM_DIM), jnp.float32),
            pltpu.SemaphoreType.DMA,
        ],
    )
    def k2(tokt_hbm, bemb_hbm, hb_hbm, tokv, bidx, brows, bsem):
        wid = lax.axis_index("s") * _INFO.num_cores + lax.axis_index("c")
        b0 = wid * BW
        _stage_tokens(tokt_hbm, tokv, b0)

        for s in range(1, S):
            for v in range(BW // LANES):
                j0 = s * BW + v * LANES
                curr = tokv[pl.ds(j0, LANES)]
                prev = tokv[pl.ds(j0 - BW, LANES)]
                h = lax.rem(lax.bitwise_xor(curr * 36313, prev * 27191),
                            jnp.int32(MOD))
                bidx[pl.ds(j0, LANES)] = h

        for r in range(8):
            pltpu.make_async_copy(bemb_hbm.at[pl.ds(MOD, 1)],
                                  brows.at[pl.ds(r, 1)], bsem).start()

        def fire(j, carry):
            idx = bidx[pl.ds(j, LANES)][0]
            pltpu.make_async_copy(bemb_hbm.at[pl.ds(idx, 1)],
                                  brows.at[pl.ds(j, 1)], bsem).start()
            return carry

        lax.fori_loop(BW, PER_W, fire, 0)
        pltpu.make_async_copy(bemb_hbm.at[pl.ds(0, PER_W - BW + 8)],
                              brows.at[pl.ds(0, PER_W - BW + 8)], bsem).wait()

        for r in range(8, BW):
            for v in range(BIGRAM_DIM // LANES):
                brows[r, pl.ds(v * LANES, LANES)] = (
                    brows[0, pl.ds(v * LANES, LANES)])

        for s in range(S):
            pltpu.sync_copy(brows.at[pl.ds(s * BW, BW)],
                            hb_hbm.at[pl.ds(s * B + b0, BW)])

    return k2(tokens_t, big_emb)


BATCH_BLK = 128


def _tc_body(tok_ref, hb_ref, emb_ref, pw_ref, bs_ref, g_ref,
             ns_ref, out_ref):
    t3 = tok_ref[...]
    oh = (lax.broadcasted_iota(jnp.int32, (S, BATCH_BLK, VOCAB), 2)
          == t3[:, :, None]).astype(jnp.float32)
    x_tok = lax.dot_general(oh, emb_ref[...],
                            (((2,), (0,)), ((), ())),
                            precision=lax.Precision.HIGHEST,
                            preferred_element_type=jnp.float32)
    hbp = lax.dot_general(hb_ref[...], pw_ref[...],
                          (((2,), (1,)), ((), ())),
                          preferred_element_type=jnp.float32)
    x = x_tok + hbp * bs_ref[0, 0]
    g = jax.nn.sigmoid(g_ref[...])
    xprev = jnp.concatenate(
        [jnp.zeros((1, BATCH_BLK, D_MODEL), jnp.float32), x[:-1]], axis=0)
    x = (1.0 - g) * x + g * xprev
    ms = jnp.mean(x * x, axis=2, keepdims=True)
    xn = x * lax.rsqrt(ms + 1e-6) * ns_ref[...]
    for s in range(S):
        lt = lax.dot_general(emb_ref[...], xn[s],
                             (((1,), (1,)), ((), ())),
                             preferred_element_type=jnp.float32)
        out_ref[s] = lt


def _tc_dense(tokens_t, hb3, tok_emb, proj_w, big_scale, gate, norm_scale):
    grid = (B // BATCH_BLK,)
    return pl.pallas_call(
        _tc_body,
        grid=grid,
        in_specs=[
            pl.BlockSpec((S, BATCH_BLK), lambda i: (0, i)),
            pl.BlockSpec((S, BATCH_BLK, BIGRAM_DIM), lambda i: (0, i, 0)),
            pl.BlockSpec((VOCAB, D_MODEL), lambda i: (0, 0)),
            pl.BlockSpec((D_MODEL, BIGRAM_DIM), lambda i: (0, 0)),
            pl.BlockSpec((1, 1), lambda i: (0, 0)),
            pl.BlockSpec((1, 1, D_MODEL), lambda i: (0, 0, 0)),
            pl.BlockSpec((1, 1, D_MODEL), lambda i: (0, 0, 0)),
        ],
        out_specs=pl.BlockSpec((S, VOCAB, BATCH_BLK), lambda i: (0, 0, i)),
        out_shape=jax.ShapeDtypeStruct((S, VOCAB, B), jnp.float32),
    )(tokens_t, hb3, tok_emb, proj_w, big_scale, gate, norm_scale)


def kernel(tokens, tok_emb, big_emb, proj_w, big_scale, gate, norm_scale):
    tokens_t = tokens.T.astype(jnp.int32)
    hb = _sc_big(tokens_t, big_emb)
    hb3 = hb.reshape(S, B, BIGRAM_DIM)
    out_t = _tc_dense(tokens_t, hb3, tok_emb, proj_w,
                      big_scale.reshape(1, 1).astype(jnp.float32),
                      gate.reshape(1, 1, D_MODEL),
                      norm_scale.reshape(1, 1, D_MODEL))
    return out_t.transpose(2, 0, 1)


# speedup vs baseline: 1.1700x; 1.1700x over previous
import functools

import jax
import jax.numpy as jnp
from jax import lax
from jax.experimental import pallas as pl
from jax.experimental.pallas import tpu as pltpu
from jax.experimental.pallas import tpu_sc as plsc

VOCAB = 1000
D_MODEL = 128
BIGRAM_VOCAB = 1000000
BIGRAM_DIM = 64
B, S = 1024, 20
N = B * S
MOD = BIGRAM_VOCAB - 1

CHUNK = 128
LANES = 16

_INFO = plsc.get_sparse_core_info()
_NW = _INFO.num_cores * _INFO.num_subcores
BW = B // _NW
PER_W = BW * S
_MESH = dict(core_axis_name="c", subcore_axis_name="s")


def _stage_tokens(tokt_hbm, tokv, b0):
    for s in range(S):
        pltpu.sync_copy(tokt_hbm.at[s, pl.ds(b0, BW)],
                        tokv.at[pl.ds(s * BW, BW)])


def _sc_big(tokens_t, big_emb):
    @functools.partial(
        pl.kernel,
        mesh=plsc.VectorSubcoreMesh(**_MESH),
        out_type=jax.ShapeDtypeStruct((N, BIGRAM_DIM), jnp.float32),
        scratch_types=[
            pltpu.VMEM((S * BW,), jnp.int32),
            pltpu.VMEM((PER_W,), jnp.int32),
            pltpu.VMEM((PER_W, BIGRAM_DIM), jnp.float32),
            pltpu.SemaphoreType.DMA,
        ],
    )
    def k2(tokt_hbm, bemb_hbm, hb_hbm, tokv, bidx, brows, bsem):
        wid = lax.axis_index("s") * _INFO.num_cores + lax.axis_index("c")
        b0 = wid * BW
        _stage_tokens(tokt_hbm, tokv, b0)

        for s in range(1, S):
            for v in range(BW // LANES):
                j0 = s * BW + v * LANES
                curr = tokv[pl.ds(j0, LANES)]
                prev = tokv[pl.ds(j0 - BW, LANES)]
                h = lax.rem(lax.bitwise_xor(curr * 36313, prev * 27191),
                            jnp.int32(MOD))
                bidx[pl.ds(j0, LANES)] = h

        for r in range(8):
            pltpu.make_async_copy(bemb_hbm.at[pl.ds(MOD, 1)],
                                  brows.at[pl.ds(r, 1)], bsem).start()

        def fire(j, carry):
            idx = bidx[pl.ds(j, LANES)][0]
            pltpu.make_async_copy(bemb_hbm.at[pl.ds(idx, 1)],
                                  brows.at[pl.ds(j, 1)], bsem).start()
            return carry

        lax.fori_loop(BW, PER_W, fire, 0)
        pltpu.make_async_copy(bemb_hbm.at[pl.ds(0, PER_W - BW + 8)],
                              brows.at[pl.ds(0, PER_W - BW + 8)], bsem).wait()

        for r in range(8, BW):
            for v in range(BIGRAM_DIM // LANES):
                brows[r, pl.ds(v * LANES, LANES)] = (
                    brows[0, pl.ds(v * LANES, LANES)])

        for s in range(S):
            pltpu.sync_copy(brows.at[pl.ds(s * BW, BW)],
                            hb_hbm.at[pl.ds(s * B + b0, BW)])

    return k2(tokens_t, big_emb)


BATCH_BLK = 128


def _tc_body(tok_ref, hb_ref, emb_hi_ref, emb_lo_ref, emb_ref, pw_ref,
             bs_ref, g_ref, ns_ref, out_ref):
    t3 = tok_ref[...]
    oh = (lax.broadcasted_iota(jnp.int32, (S, BATCH_BLK, VOCAB), 2)
          == t3[:, :, None]).astype(jnp.float32)
    dn = (((2,), (0,)), ((), ()))
    x_tok = (lax.dot_general(oh, emb_hi_ref[...], dn,
                             preferred_element_type=jnp.float32)
             + lax.dot_general(oh, emb_lo_ref[...], dn,
                               preferred_element_type=jnp.float32))
    hbp = lax.dot_general(hb_ref[...], pw_ref[...],
                          (((2,), (1,)), ((), ())),
                          preferred_element_type=jnp.float32)
    x = x_tok + hbp * bs_ref[0, 0]
    g = jax.nn.sigmoid(g_ref[...])
    xprev = jnp.concatenate(
        [jnp.zeros((1, BATCH_BLK, D_MODEL), jnp.float32), x[:-1]], axis=0)
    x = (1.0 - g) * x + g * xprev
    ms = jnp.mean(x * x, axis=2, keepdims=True)
    xn = x * lax.rsqrt(ms + 1e-6) * ns_ref[...]
    for s in range(S):
        lt = lax.dot_general(emb_ref[...], xn[s],
                             (((1,), (1,)), ((), ())),
                             preferred_element_type=jnp.float32)
        out_ref[s] = lt


def _tc_dense(tokens_t, hb3, emb_hi, emb_lo, tok_emb, proj_w, big_scale,
              gate, norm_scale):
    grid = (B // BATCH_BLK,)
    return pl.pallas_call(
        _tc_body,
        grid=grid,
        in_specs=[
            pl.BlockSpec((S, BATCH_BLK), lambda i: (0, i)),
            pl.BlockSpec((S, BATCH_BLK, BIGRAM_DIM), lambda i: (0, i, 0)),
            pl.BlockSpec((VOCAB, D_MODEL), lambda i: (0, 0)),
            pl.BlockSpec((VOCAB, D_MODEL), lambda i: (0, 0)),
            pl.BlockSpec((VOCAB, D_MODEL), lambda i: (0, 0)),
            pl.BlockSpec((D_MODEL, BIGRAM_DIM), lambda i: (0, 0)),
            pl.BlockSpec((1, 1), lambda i: (0, 0)),
            pl.BlockSpec((1, 1, D_MODEL), lambda i: (0, 0, 0)),
            pl.BlockSpec((1, 1, D_MODEL), lambda i: (0, 0, 0)),
        ],
        out_specs=pl.BlockSpec((S, VOCAB, BATCH_BLK), lambda i: (0, 0, i)),
        out_shape=jax.ShapeDtypeStruct((S, VOCAB, B), jnp.float32),
    )(tokens_t, hb3, emb_hi, emb_lo, tok_emb, proj_w, big_scale, gate,
      norm_scale)


def kernel(tokens, tok_emb, big_emb, proj_w, big_scale, gate, norm_scale):
    tokens_t = tokens.T.astype(jnp.int32)
    hb = _sc_big(tokens_t, big_emb)
    hb3 = hb.reshape(S, B, BIGRAM_DIM)
    emb_hi = tok_emb.astype(jnp.bfloat16).astype(jnp.float32)
    emb_lo = tok_emb - emb_hi
    out_t = _tc_dense(tokens_t, hb3, emb_hi, emb_lo, tok_emb, proj_w,
                      big_scale.reshape(1, 1).astype(jnp.float32),
                      gate.reshape(1, 1, D_MODEL),
                      norm_scale.reshape(1, 1, D_MODEL))
    return out_t.transpose(2, 0, 1)


# speedup vs baseline: 1.1785x; 1.0072x over previous
import functools

import jax
import jax.numpy as jnp
from jax import lax
from jax.experimental import pallas as pl
from jax.experimental.pallas import tpu as pltpu
from jax.experimental.pallas import tpu_sc as plsc

VOCAB = 1000
D_MODEL = 128
BIGRAM_VOCAB = 1000000
BIGRAM_DIM = 64
B, S = 1024, 20
N = B * S
MOD = BIGRAM_VOCAB - 1

CHUNK = 128
LANES = 16

_INFO = plsc.get_sparse_core_info()
_NW = _INFO.num_cores * _INFO.num_subcores
BW = B // _NW
PER_W = BW * S
_MESH = dict(core_axis_name="c", subcore_axis_name="s")


def _stage_tokens(tokt_hbm, tokv, b0):
    for s in range(S):
        pltpu.sync_copy(tokt_hbm.at[s, pl.ds(b0, BW)],
                        tokv.at[pl.ds(s * BW, BW)])


def _sc_big(tokens_t, big_emb):
    @functools.partial(
        pl.kernel,
        mesh=plsc.VectorSubcoreMesh(**_MESH),
        out_type=jax.ShapeDtypeStruct((N, BIGRAM_DIM), jnp.float32),
        scratch_types=[
            pltpu.VMEM((S * BW,), jnp.int32),
            pltpu.VMEM((PER_W,), jnp.int32),
            pltpu.VMEM((PER_W, BIGRAM_DIM), jnp.float32),
            pltpu.SemaphoreType.DMA,
        ],
    )
    def k2(tokt_hbm, bemb_hbm, hb_hbm, tokv, bidx, brows, bsem):
        wid = lax.axis_index("s") * _INFO.num_cores + lax.axis_index("c")
        b0 = wid * BW
        _stage_tokens(tokt_hbm, tokv, b0)

        for s in range(1, S):
            for v in range(BW // LANES):
                j0 = s * BW + v * LANES
                curr = tokv[pl.ds(j0, LANES)]
                prev = tokv[pl.ds(j0 - BW, LANES)]
                h = lax.rem(lax.bitwise_xor(curr * 36313, prev * 27191),
                            jnp.int32(MOD))
                bidx[pl.ds(j0, LANES)] = h

        for r in range(8):
            pltpu.make_async_copy(bemb_hbm.at[pl.ds(MOD, 1)],
                                  brows.at[pl.ds(r, 1)], bsem).start()

        def fire(j, carry):
            idx = bidx[pl.ds(j, LANES)][0]
            pltpu.make_async_copy(bemb_hbm.at[pl.ds(idx, 1)],
                                  brows.at[pl.ds(j, 1)], bsem).start()
            return carry

        lax.fori_loop(BW, PER_W, fire, 0)
        pltpu.make_async_copy(bemb_hbm.at[pl.ds(0, PER_W - BW + 8)],
                              brows.at[pl.ds(0, PER_W - BW + 8)], bsem).wait()

        for r in range(8, BW):
            for v in range(BIGRAM_DIM // LANES):
                brows[r, pl.ds(v * LANES, LANES)] = (
                    brows[0, pl.ds(v * LANES, LANES)])

        for s in range(S):
            pltpu.sync_copy(brows.at[pl.ds(s * BW, BW)],
                            hb_hbm.at[pl.ds(s * B + b0, BW)])

    return k2(tokens_t, big_emb)


BATCH_BLK = 128


def _tc_body(tok_ref, hb_ref, emb_ref, pw_ref,
             bs_ref, g_ref, ns_ref, out_ref):
    t3 = tok_ref[...]
    oh = (lax.broadcasted_iota(jnp.int32, (S, BATCH_BLK, VOCAB), 2)
          == t3[:, :, None]).astype(jnp.float32)
    x_tok = lax.dot_general(oh, emb_ref[...],
                            (((2,), (0,)), ((), ())),
                            preferred_element_type=jnp.float32)
    hbp = lax.dot_general(hb_ref[...], pw_ref[...],
                          (((2,), (1,)), ((), ())),
                          preferred_element_type=jnp.float32)
    x = x_tok + hbp * bs_ref[0, 0]
    g = jax.nn.sigmoid(g_ref[...])
    xprev = jnp.concatenate(
        [jnp.zeros((1, BATCH_BLK, D_MODEL), jnp.float32), x[:-1]], axis=0)
    x = (1.0 - g) * x + g * xprev
    ms = jnp.mean(x * x, axis=2, keepdims=True)
    xn = x * lax.rsqrt(ms + 1e-6) * ns_ref[...]
    for s in range(S):
        lt = lax.dot_general(emb_ref[...], xn[s],
                             (((1,), (1,)), ((), ())),
                             preferred_element_type=jnp.float32)
        out_ref[s] = lt


def _tc_dense(tokens_t, hb3, tok_emb, proj_w, big_scale, gate, norm_scale):
    grid = (B // BATCH_BLK,)
    return pl.pallas_call(
        _tc_body,
        grid=grid,
        in_specs=[
            pl.BlockSpec((S, BATCH_BLK), lambda i: (0, i)),
            pl.BlockSpec((S, BATCH_BLK, BIGRAM_DIM), lambda i: (0, i, 0)),
            pl.BlockSpec((VOCAB, D_MODEL), lambda i: (0, 0)),
            pl.BlockSpec((D_MODEL, BIGRAM_DIM), lambda i: (0, 0)),
            pl.BlockSpec((1, 1), lambda i: (0, 0)),
            pl.BlockSpec((1, 1, D_MODEL), lambda i: (0, 0, 0)),
            pl.BlockSpec((1, 1, D_MODEL), lambda i: (0, 0, 0)),
        ],
        out_specs=pl.BlockSpec((S, VOCAB, BATCH_BLK), lambda i: (0, 0, i)),
        out_shape=jax.ShapeDtypeStruct((S, VOCAB, B), jnp.float32),
    )(tokens_t, hb3, tok_emb, proj_w, big_scale, gate, norm_scale)


def kernel(tokens, tok_emb, big_emb, proj_w, big_scale, gate, norm_scale):
    tokens_t = tokens.T.astype(jnp.int32)
    hb = _sc_big(tokens_t, big_emb)
    hb3 = hb.reshape(S, B, BIGRAM_DIM)
    out_t = _tc_dense(tokens_t, hb3, tok_emb, proj_w,
                      big_scale.reshape(1, 1).astype(jnp.float32),
                      gate.reshape(1, 1, D_MODEL),
                      norm_scale.reshape(1, 1, D_MODEL))
    return out_t.transpose(2, 0, 1)
